# baseline (speedup 1.0000x reference)
import functools

import jax
import jax.numpy as jnp
from jax import lax
from jax.experimental import pallas as pl
from jax.experimental.pallas import tpu as pltpu
from jax.experimental.pallas import tpu_sc as plsc

_K = 8192
_D = 32
_B = 8192
_BETA = 0.25
_BB = 256
_W = 2048
_NW = 32
_BPW = _B // _NW


def _rownorm(x):
    sq = x * x
    p = ((sq[:, 8:16] + sq[:, 0:8]) + sq[:, 16:24]) + sq[:, 24:32]
    q = p[:, 4:8] + p[:, 0:4]
    r = q[:, 2:4] + q[:, 0:2]
    return r[:, 1:2] + r[:, 0:1]


def _argmin_body(z_ref, cb_ref, idx_ref, loss_ref):
    z = z_ref[...]
    cb = cb_ref[...]
    cnorm = _rownorm(cb)[:, 0]
    znorm = _rownorm(z)
    zc = lax.dot_general(z, cb, (((1,), (1,)), ((), ())),
                         preferred_element_type=jnp.float32)
    d2 = znorm - 2.0 * zc + cnorm[None, :]
    iota = lax.broadcasted_iota(jnp.int32, d2.shape, 1)
    best_v = best_i = None
    for c in range(_K // _W):
        blk = d2[:, c * _W:(c + 1) * _W]
        ib = iota[:, c * _W:(c + 1) * _W]
        m = jnp.min(blk, axis=1)
        i = jnp.min(jnp.where(blk == m[:, None], ib, _K), axis=1)
        mb = m.astype(jnp.bfloat16).astype(jnp.float32)
        if best_v is None:
            best_v, best_i = mb, i
        else:
            take = (m < best_v) | ((m == best_v) & (i < best_i))
            best_v = jnp.where(take, mb, best_v)
            best_i = jnp.where(take, i, best_i)
    idx_ref[...] = best_i
    loss_row = jnp.min(jnp.where(iota == best_i[:, None], d2, jnp.inf),
                       axis=1)

    @pl.when(pl.program_id(0) == 0)
    def _():
        loss_ref[0] = 0.0

    loss_ref[0] += jnp.sum(loss_row)


def _argmin_call(z_e, codebook):
    return pl.pallas_call(
        _argmin_body,
        grid=(_B // _BB,),
        in_specs=[
            pl.BlockSpec((_BB, _D), lambda i: (i, 0)),
            pl.BlockSpec((_K, _D), lambda i: (0, 0)),
        ],
        out_specs=[
            pl.BlockSpec((_BB,), lambda i: (i,)),
            pl.BlockSpec(memory_space=pltpu.SMEM),
        ],
        out_shape=[
            jax.ShapeDtypeStruct((_B,), jnp.int32),
            jax.ShapeDtypeStruct((1,), jnp.float32),
        ],
        compiler_params=pltpu.CompilerParams(
            dimension_semantics=("arbitrary",)),
    )(z_e, codebook)


@functools.cache
def _make_gather_rows():
    @functools.partial(
        pl.kernel,
        mesh=plsc.VectorSubcoreMesh(core_axis_name="c", subcore_axis_name="s"),
        out_type=jax.ShapeDtypeStruct((_B, _D), jnp.float32),
        scratch_types=[
            pltpu.VMEM((_BPW,), jnp.int32),
            pltpu.VMEM((_BPW, _D), jnp.float32),
            pltpu.SemaphoreType.DMA,
        ],
        compiler_params=pltpu.CompilerParams(use_tc_tiling_on_sc=False),
    )
    def _gather_rows(cb_hbm, idx_hbm, out_hbm, idx_v, rows_v, sem):
        wid = lax.axis_index("s") * 2 + lax.axis_index("c")
        base = wid * _BPW
        pltpu.sync_copy(idx_hbm.at[pl.ds(base, _BPW)], idx_v)
        pltpu.async_copy(cb_hbm.at[idx_v], rows_v, sem).wait()
        pltpu.sync_copy(rows_v, out_hbm.at[pl.ds(base, _BPW)])

    return _gather_rows


def kernel(z_e, codebook):
    idx, loss_sum = _argmin_call(z_e, codebook)
    z_q = _make_gather_rows()(codebook, idx)
    vq_loss = (1.0 + _BETA) * loss_sum[0] / (_B * _D)
    return (z_q, idx, vq_loss)


# speedup vs baseline: 1.2867x; 1.2867x over previous
import functools

import jax
import jax.numpy as jnp
from jax import lax
from jax.experimental import pallas as pl
from jax.experimental.pallas import tpu as pltpu
from jax.experimental.pallas import tpu_sc as plsc

_K = 8192
_D = 32
_B = 8192
_BETA = 0.25
_BB = 256
_W = 2048
_NW = 32
_BPW = _B // _NW


def _rownorm(x):
    sq = x * x
    p = ((sq[:, 8:16] + sq[:, 0:8]) + sq[:, 16:24]) + sq[:, 24:32]
    q = p[:, 4:8] + p[:, 0:4]
    r = q[:, 2:4] + q[:, 0:2]
    return r[:, 1:2] + r[:, 0:1]


def _colnorm(xt):
    sq = xt * xt
    p = ((sq[8:16, :] + sq[0:8, :]) + sq[16:24, :]) + sq[24:32, :]
    q = p[4:8, :] + p[0:4, :]
    r = q[2:4, :] + q[0:2, :]
    return r[1:2, :] + r[0:1, :]


def _argmin_body(z_ref, cb_ref, idx_ref, loss_ref):
    zt = z_ref[...].T
    znorm = _colnorm(zt)
    best_v = best_i = loss_m = None
    for c in range(_K // _W):
        cbw = cb_ref[pl.ds(c * _W, _W), :]
        cw2 = cbw + cbw
        cnorm = _rownorm(cbw)
        zc2 = lax.dot_general(cw2, zt, (((1,), (0,)), ((), ())),
                              preferred_element_type=jnp.float32)
        d2 = (znorm - zc2) + cnorm
        m = jnp.min(d2, axis=0)
        iota = lax.broadcasted_iota(jnp.int32, (_W, _BB), 0)
        il = jnp.min(jnp.where(d2 == m[None, :], iota, _W), axis=0)
        i = il + (c * _W)
        mb = m.astype(jnp.bfloat16).astype(jnp.float32)
        if best_v is None:
            best_v, best_i, loss_m = mb, i, m
        else:
            take = (m < best_v) | ((m == best_v) & (i < best_i))
            best_v = jnp.where(take, mb, best_v)
            best_i = jnp.where(take, i, best_i)
            loss_m = jnp.minimum(loss_m, m)
    idx_ref[...] = best_i

    @pl.when(pl.program_id(0) == 0)
    def _():
        loss_ref[0] = 0.0

    loss_ref[0] += jnp.sum(loss_m)


def _argmin_call(z_e, codebook):
    return pl.pallas_call(
        _argmin_body,
        grid=(_B // _BB,),
        in_specs=[
            pl.BlockSpec((_BB, _D), lambda i: (i, 0)),
            pl.BlockSpec((_K, _D), lambda i: (0, 0)),
        ],
        out_specs=[
            pl.BlockSpec((_BB,), lambda i: (i,)),
            pl.BlockSpec(memory_space=pltpu.SMEM),
        ],
        out_shape=[
            jax.ShapeDtypeStruct((_B,), jnp.int32),
            jax.ShapeDtypeStruct((1,), jnp.float32),
        ],
        compiler_params=pltpu.CompilerParams(
            dimension_semantics=("arbitrary",)),
    )(z_e, codebook)


@functools.cache
def _make_gather_rows():
    @functools.partial(
        pl.kernel,
        mesh=plsc.VectorSubcoreMesh(core_axis_name="c", subcore_axis_name="s"),
        out_type=jax.ShapeDtypeStruct((_B, _D), jnp.float32),
        scratch_types=[
            pltpu.VMEM((_BPW,), jnp.int32),
            pltpu.VMEM((_BPW, _D), jnp.float32),
            pltpu.SemaphoreType.DMA,
        ],
        compiler_params=pltpu.CompilerParams(use_tc_tiling_on_sc=False),
    )
    def _gather_rows(cb_hbm, idx_hbm, out_hbm, idx_v, rows_v, sem):
        wid = lax.axis_index("s") * 2 + lax.axis_index("c")
        base = wid * _BPW
        pltpu.sync_copy(idx_hbm.at[pl.ds(base, _BPW)], idx_v)
        pltpu.async_copy(cb_hbm.at[idx_v], rows_v, sem).wait()
        pltpu.sync_copy(rows_v, out_hbm.at[pl.ds(base, _BPW)])

    return _gather_rows


def kernel(z_e, codebook):
    idx, loss_sum = _argmin_call(z_e, codebook)
    z_q = _make_gather_rows()(codebook, idx)
    vq_loss = (1.0 + _BETA) * loss_sum[0] / (_B * _D)
    return (z_q, idx, vq_loss)


# speedup vs baseline: 2.9288x; 2.2762x over previous
import functools

import jax
import jax.numpy as jnp
from jax import lax
from jax.experimental import pallas as pl
from jax.experimental.pallas import tpu as pltpu
from jax.experimental.pallas import tpu_sc as plsc

_K = 8192
_D = 32
_B = 8192
_BETA = 0.25
_BB = 256
_W = 2048
_NW = 32
_BPW = _B // _NW


def _rownorm(x):
    sq = x * x
    p = ((sq[:, 8:16] + sq[:, 0:8]) + sq[:, 16:24]) + sq[:, 24:32]
    q = p[:, 4:8] + p[:, 0:4]
    r = q[:, 2:4] + q[:, 0:2]
    return r[:, 1:2] + r[:, 0:1]


def _colnorm(xt):
    sq = xt * xt
    p = ((sq[8:16, :] + sq[0:8, :]) + sq[16:24, :]) + sq[24:32, :]
    q = p[4:8, :] + p[0:4, :]
    r = q[2:4, :] + q[0:2, :]
    return r[1:2, :] + r[0:1, :]


def _prep_body(cb_ref, cb2_ref, cnorm_ref):
    cb = cb_ref[...]
    cb2_ref[...] = cb + cb
    cnorm_ref[...] = _rownorm(cb)


def _prep_call(codebook):
    return pl.pallas_call(
        _prep_body,
        out_shape=[
            jax.ShapeDtypeStruct((_K, _D), jnp.float32),
            jax.ShapeDtypeStruct((_K, 1), jnp.float32),
        ],
    )(codebook)


def _argmin_body(z_ref, cb2_ref, cn_ref, idx_ref, loss_ref):
    zt = z_ref[...].T
    znorm = _colnorm(zt)
    best_v = best_i = loss_m = None
    for c in range(_K // _W):
        cw2 = cb2_ref[pl.ds(c * _W, _W), :]
        cnorm = cn_ref[pl.ds(c * _W, _W), :]
        zc2 = lax.dot_general(cw2, zt, (((1,), (0,)), ((), ())),
                              preferred_element_type=jnp.float32)
        d2 = (znorm - zc2) + cnorm
        m = jnp.min(d2, axis=0)
        iota = lax.broadcasted_iota(jnp.int32, (_W, _BB), 0)
        il = jnp.min(jnp.where(d2 == m[None, :], iota, _W), axis=0)
        i = il + (c * _W)
        mb = m.astype(jnp.bfloat16).astype(jnp.float32)
        if best_v is None:
            best_v, best_i, loss_m = mb, i, m
        else:
            take = (m < best_v) | ((m == best_v) & (i < best_i))
            best_v = jnp.where(take, mb, best_v)
            best_i = jnp.where(take, i, best_i)
            loss_m = jnp.minimum(loss_m, m)
    idx_ref[...] = best_i

    @pl.when(pl.program_id(0) == 0)
    def _():
        loss_ref[0] = 0.0

    loss_ref[0] += jnp.sum(loss_m)


def _argmin_call(z_e, cb2, cnorm):
    return pl.pallas_call(
        _argmin_body,
        grid=(_B // _BB,),
        in_specs=[
            pl.BlockSpec((_BB, _D), lambda i: (i, 0)),
            pl.BlockSpec((_K, _D), lambda i: (0, 0)),
            pl.BlockSpec((_K, 1), lambda i: (0, 0)),
        ],
        out_specs=[
            pl.BlockSpec((_BB,), lambda i: (i,)),
            pl.BlockSpec(memory_space=pltpu.SMEM),
        ],
        out_shape=[
            jax.ShapeDtypeStruct((_B,), jnp.int32),
            jax.ShapeDtypeStruct((1,), jnp.float32),
        ],
        compiler_params=pltpu.CompilerParams(
            dimension_semantics=("arbitrary",)),
    )(z_e, cb2, cnorm)


@functools.cache
def _make_gather_rows():
    @functools.partial(
        pl.kernel,
        mesh=plsc.VectorSubcoreMesh(core_axis_name="c", subcore_axis_name="s"),
        out_type=jax.ShapeDtypeStruct((_B, _D), jnp.float32),
        scratch_types=[
            pltpu.VMEM((_BPW,), jnp.int32),
            pltpu.VMEM((_BPW, _D), jnp.float32),
            pltpu.SemaphoreType.DMA,
        ],
        compiler_params=pltpu.CompilerParams(use_tc_tiling_on_sc=False),
    )
    def _gather_rows(cb_hbm, idx_hbm, out_hbm, idx_v, rows_v, sem):
        wid = lax.axis_index("s") * 2 + lax.axis_index("c")
        base = wid * _BPW
        pltpu.sync_copy(idx_hbm.at[pl.ds(base, _BPW)], idx_v)
        pltpu.async_copy(cb_hbm.at[idx_v], rows_v, sem).wait()
        pltpu.sync_copy(rows_v, out_hbm.at[pl.ds(base, _BPW)])

    return _gather_rows


def kernel(z_e, codebook):
    cb2, cnorm = _prep_call(codebook)
    idx, loss_sum = _argmin_call(z_e, cb2, cnorm)
    z_q = _make_gather_rows()(codebook, idx)
    vq_loss = (1.0 + _BETA) * loss_sum[0] / (_B * _D)
    return (z_q, idx, vq_loss)


# speedup vs baseline: 3.0422x; 1.0387x over previous
import functools

import jax
import jax.numpy as jnp
from jax import lax
from jax.experimental import pallas as pl
from jax.experimental.pallas import tpu as pltpu
from jax.experimental.pallas import tpu_sc as plsc

_K = 8192
_D = 32
_B = 8192
_BETA = 0.25
_BB = 256
_W = 2048
_NW = 32
_BPW = _B // _NW


def _rownorm(x):
    sq = x * x
    p = ((sq[:, 8:16] + sq[:, 0:8]) + sq[:, 16:24]) + sq[:, 24:32]
    q = p[:, 4:8] + p[:, 0:4]
    r = q[:, 2:4] + q[:, 0:2]
    return r[:, 1:2] + r[:, 0:1]


def _colnorm(xt):
    sq = xt * xt
    p = ((sq[8:16, :] + sq[0:8, :]) + sq[16:24, :]) + sq[24:32, :]
    q = p[4:8, :] + p[0:4, :]
    r = q[2:4, :] + q[0:2, :]
    return r[1:2, :] + r[0:1, :]


def _prep_body(cb_ref, cb2_ref, cnorm_ref):
    cb = cb_ref[...]
    cb2_ref[...] = cb + cb
    cnorm_ref[...] = _colnorm(cb.T).T


def _prep_call(codebook):
    return pl.pallas_call(
        _prep_body,
        out_shape=[
            jax.ShapeDtypeStruct((_K, _D), jnp.float32),
            jax.ShapeDtypeStruct((_K, 1), jnp.float32),
        ],
    )(codebook)


def _argmin_body(z_ref, cb2_ref, cn_ref, idx_ref, loss_ref):
    zt = z_ref[...].T
    znorm = _colnorm(zt)
    best_v = best_i = loss_m = None
    for c in range(_K // _W):
        cw2 = cb2_ref[pl.ds(c * _W, _W), :]
        cnorm = cn_ref[pl.ds(c * _W, _W), :]
        zc2 = lax.dot_general(cw2, zt, (((1,), (0,)), ((), ())),
                              preferred_element_type=jnp.float32)
        d2 = (znorm - zc2) + cnorm
        m = jnp.min(d2, axis=0)
        iota = lax.broadcasted_iota(jnp.int32, (_W, _BB), 0)
        il = jnp.min(jnp.where(d2 == m[None, :], iota, _W), axis=0)
        i = il + (c * _W)
        mb = m.astype(jnp.bfloat16).astype(jnp.float32)
        if best_v is None:
            best_v, best_i, loss_m = mb, i, m
        else:
            take = (m < best_v) | ((m == best_v) & (i < best_i))
            best_v = jnp.where(take, mb, best_v)
            best_i = jnp.where(take, i, best_i)
            loss_m = jnp.minimum(loss_m, m)
    idx_ref[...] = best_i

    @pl.when(pl.program_id(0) == 0)
    def _():
        loss_ref[0] = 0.0

    loss_ref[0] += jnp.sum(loss_m)


def _argmin_call(z_e, cb2, cnorm):
    return pl.pallas_call(
        _argmin_body,
        grid=(_B // _BB,),
        in_specs=[
            pl.BlockSpec((_BB, _D), lambda i: (i, 0)),
            pl.BlockSpec((_K, _D), lambda i: (0, 0)),
            pl.BlockSpec((_K, 1), lambda i: (0, 0)),
        ],
        out_specs=[
            pl.BlockSpec((_BB,), lambda i: (i,)),
            pl.BlockSpec(memory_space=pltpu.SMEM),
        ],
        out_shape=[
            jax.ShapeDtypeStruct((_B,), jnp.int32),
            jax.ShapeDtypeStruct((1,), jnp.float32),
        ],
        compiler_params=pltpu.CompilerParams(
            dimension_semantics=("arbitrary",)),
    )(z_e, cb2, cnorm)


@functools.cache
def _make_gather_rows():
    @functools.partial(
        pl.kernel,
        mesh=plsc.VectorSubcoreMesh(core_axis_name="c", subcore_axis_name="s"),
        out_type=jax.ShapeDtypeStruct((_B, _D), jnp.float32),
        scratch_types=[
            pltpu.VMEM((_BPW,), jnp.int32),
            pltpu.VMEM((_BPW, _D), jnp.float32),
            pltpu.SemaphoreType.DMA,
        ],
        compiler_params=pltpu.CompilerParams(use_tc_tiling_on_sc=False),
    )
    def _gather_rows(cb_hbm, idx_hbm, out_hbm, idx_v, rows_v, sem):
        wid = lax.axis_index("s") * 2 + lax.axis_index("c")
        base = wid * _BPW
        pltpu.sync_copy(idx_hbm.at[pl.ds(base, _BPW)], idx_v)
        pltpu.async_copy(cb_hbm.at[idx_v], rows_v, sem).wait()
        pltpu.sync_copy(rows_v, out_hbm.at[pl.ds(base, _BPW)])

    return _gather_rows


def kernel(z_e, codebook):
    cb2, cnorm = _prep_call(codebook)
    idx, loss_sum = _argmin_call(z_e, cb2, cnorm)
    z_q = _make_gather_rows()(codebook, idx)
    vq_loss = (1.0 + _BETA) * loss_sum[0] / (_B * _D)
    return (z_q, idx, vq_loss)


# speedup vs baseline: 3.2343x; 1.0632x over previous
import functools

import jax
import jax.numpy as jnp
from jax import lax
from jax.experimental import pallas as pl
from jax.experimental.pallas import tpu as pltpu
from jax.experimental.pallas import tpu_sc as plsc

_K = 8192
_D = 32
_B = 8192
_BETA = 0.25
_BB = 512
_W = 2048
_NW = 32
_BPW = _B // _NW


def _rownorm(x):
    sq = x * x
    p = ((sq[:, 8:16] + sq[:, 0:8]) + sq[:, 16:24]) + sq[:, 24:32]
    q = p[:, 4:8] + p[:, 0:4]
    r = q[:, 2:4] + q[:, 0:2]
    return r[:, 1:2] + r[:, 0:1]


def _colnorm(xt):
    sq = xt * xt
    p = ((sq[8:16, :] + sq[0:8, :]) + sq[16:24, :]) + sq[24:32, :]
    q = p[4:8, :] + p[0:4, :]
    r = q[2:4, :] + q[0:2, :]
    return r[1:2, :] + r[0:1, :]


def _prep_body(cb_ref, cb2_ref, cnorm_ref):
    cb = cb_ref[...]
    cb2_ref[...] = cb + cb
    cnorm_ref[...] = _colnorm(cb.T).T


def _prep_call(codebook):
    return pl.pallas_call(
        _prep_body,
        out_shape=[
            jax.ShapeDtypeStruct((_K, _D), jnp.float32),
            jax.ShapeDtypeStruct((_K, 1), jnp.float32),
        ],
    )(codebook)


def _argmin_body(z_ref, cb2_ref, cn_ref, idx_ref, loss_ref):
    zt = z_ref[...].T
    znorm = _colnorm(zt)
    best_v = best_i = loss_m = None
    for c in range(_K // _W):
        cw2 = cb2_ref[pl.ds(c * _W, _W), :]
        cnorm = cn_ref[pl.ds(c * _W, _W), :]
        zc2 = lax.dot_general(cw2, zt, (((1,), (0,)), ((), ())),
                              preferred_element_type=jnp.float32)
        d2 = (znorm - zc2) + cnorm
        m = jnp.min(d2, axis=0)
        iota = lax.broadcasted_iota(jnp.int32, (_W, _BB), 0)
        il = jnp.min(jnp.where(d2 == m[None, :], iota, _W), axis=0)
        i = il + (c * _W)
        mb = m.astype(jnp.bfloat16).astype(jnp.float32)
        if best_v is None:
            best_v, best_i, loss_m = mb, i, m
        else:
            take = (m < best_v) | ((m == best_v) & (i < best_i))
            best_v = jnp.where(take, mb, best_v)
            best_i = jnp.where(take, i, best_i)
            loss_m = jnp.minimum(loss_m, m)
    idx_ref[...] = best_i

    @pl.when(pl.program_id(0) == 0)
    def _():
        loss_ref[0] = 0.0

    loss_ref[0] += jnp.sum(loss_m)


def _argmin_call(z_e, cb2, cnorm):
    return pl.pallas_call(
        _argmin_body,
        grid=(_B // _BB,),
        in_specs=[
            pl.BlockSpec((_BB, _D), lambda i: (i, 0)),
            pl.BlockSpec((_K, _D), lambda i: (0, 0)),
            pl.BlockSpec((_K, 1), lambda i: (0, 0)),
        ],
        out_specs=[
            pl.BlockSpec((_BB,), lambda i: (i,)),
            pl.BlockSpec(memory_space=pltpu.SMEM),
        ],
        out_shape=[
            jax.ShapeDtypeStruct((_B,), jnp.int32),
            jax.ShapeDtypeStruct((1,), jnp.float32),
        ],
        compiler_params=pltpu.CompilerParams(
            dimension_semantics=("arbitrary",)),
    )(z_e, cb2, cnorm)


@functools.cache
def _make_gather_rows():
    @functools.partial(
        pl.kernel,
        mesh=plsc.VectorSubcoreMesh(core_axis_name="c", subcore_axis_name="s"),
        out_type=jax.ShapeDtypeStruct((_B, _D), jnp.float32),
        scratch_types=[
            pltpu.VMEM((_BPW,), jnp.int32),
            pltpu.VMEM((_BPW, _D), jnp.float32),
            pltpu.SemaphoreType.DMA,
        ],
        compiler_params=pltpu.CompilerParams(use_tc_tiling_on_sc=False),
    )
    def _gather_rows(cb_hbm, idx_hbm, out_hbm, idx_v, rows_v, sem):
        wid = lax.axis_index("s") * 2 + lax.axis_index("c")
        base = wid * _BPW
        pltpu.sync_copy(idx_hbm.at[pl.ds(base, _BPW)], idx_v)
        pltpu.async_copy(cb_hbm.at[idx_v], rows_v, sem).wait()
        pltpu.sync_copy(rows_v, out_hbm.at[pl.ds(base, _BPW)])

    return _gather_rows


def kernel(z_e, codebook):
    cb2, cnorm = _prep_call(codebook)
    idx, loss_sum = _argmin_call(z_e, cb2, cnorm)
    z_q = _make_gather_rows()(codebook, idx)
    vq_loss = (1.0 + _BETA) * loss_sum[0] / (_B * _D)
    return (z_q, idx, vq_loss)


# speedup vs baseline: 3.2663x; 1.0099x over previous
import functools

import jax
import jax.numpy as jnp
from jax import lax
from jax.experimental import pallas as pl
from jax.experimental.pallas import tpu as pltpu
from jax.experimental.pallas import tpu_sc as plsc

_K = 8192
_D = 32
_B = 8192
_BETA = 0.25
_BB = 1024
_W = 2048
_NW = 32
_BPW = _B // _NW


def _rownorm(x):
    sq = x * x
    p = ((sq[:, 8:16] + sq[:, 0:8]) + sq[:, 16:24]) + sq[:, 24:32]
    q = p[:, 4:8] + p[:, 0:4]
    r = q[:, 2:4] + q[:, 0:2]
    return r[:, 1:2] + r[:, 0:1]


def _colnorm(xt):
    sq = xt * xt
    p = ((sq[8:16, :] + sq[0:8, :]) + sq[16:24, :]) + sq[24:32, :]
    q = p[4:8, :] + p[0:4, :]
    r = q[2:4, :] + q[0:2, :]
    return r[1:2, :] + r[0:1, :]


def _prep_body(cb_ref, cb2_ref, cnorm_ref):
    cb = cb_ref[...]
    cb2_ref[...] = cb + cb
    cnorm_ref[...] = _colnorm(cb.T).T


def _prep_call(codebook):
    return pl.pallas_call(
        _prep_body,
        out_shape=[
            jax.ShapeDtypeStruct((_K, _D), jnp.float32),
            jax.ShapeDtypeStruct((_K, 1), jnp.float32),
        ],
    )(codebook)


def _argmin_body(z_ref, cb2_ref, cn_ref, idx_ref, loss_ref):
    zt = z_ref[...].T
    znorm = _colnorm(zt)
    best_v = best_i = loss_m = None
    for c in range(_K // _W):
        cw2 = cb2_ref[pl.ds(c * _W, _W), :]
        cnorm = cn_ref[pl.ds(c * _W, _W), :]
        zc2 = lax.dot_general(cw2, zt, (((1,), (0,)), ((), ())),
                              preferred_element_type=jnp.float32)
        d2 = (znorm - zc2) + cnorm
        m = jnp.min(d2, axis=0)
        iota = lax.broadcasted_iota(jnp.int32, (_W, _BB), 0)
        il = jnp.min(jnp.where(d2 == m[None, :], iota, _W), axis=0)
        i = il + (c * _W)
        mb = m.astype(jnp.bfloat16).astype(jnp.float32)
        if best_v is None:
            best_v, best_i, loss_m = mb, i, m
        else:
            take = (m < best_v) | ((m == best_v) & (i < best_i))
            best_v = jnp.where(take, mb, best_v)
            best_i = jnp.where(take, i, best_i)
            loss_m = jnp.minimum(loss_m, m)
    idx_ref[...] = best_i

    @pl.when(pl.program_id(0) == 0)
    def _():
        loss_ref[0] = 0.0

    loss_ref[0] += jnp.sum(loss_m)


def _argmin_call(z_e, cb2, cnorm):
    return pl.pallas_call(
        _argmin_body,
        grid=(_B // _BB,),
        in_specs=[
            pl.BlockSpec((_BB, _D), lambda i: (i, 0)),
            pl.BlockSpec((_K, _D), lambda i: (0, 0)),
            pl.BlockSpec((_K, 1), lambda i: (0, 0)),
        ],
        out_specs=[
            pl.BlockSpec((_BB,), lambda i: (i,)),
            pl.BlockSpec(memory_space=pltpu.SMEM),
        ],
        out_shape=[
            jax.ShapeDtypeStruct((_B,), jnp.int32),
            jax.ShapeDtypeStruct((1,), jnp.float32),
        ],
        compiler_params=pltpu.CompilerParams(
            dimension_semantics=("arbitrary",)),
    )(z_e, cb2, cnorm)


@functools.cache
def _make_gather_rows():
    @functools.partial(
        pl.kernel,
        mesh=plsc.VectorSubcoreMesh(core_axis_name="c", subcore_axis_name="s"),
        out_type=jax.ShapeDtypeStruct((_B, _D), jnp.float32),
        scratch_types=[
            pltpu.VMEM((_BPW,), jnp.int32),
            pltpu.VMEM((_BPW, _D), jnp.float32),
            pltpu.SemaphoreType.DMA,
        ],
        compiler_params=pltpu.CompilerParams(use_tc_tiling_on_sc=False),
    )
    def _gather_rows(cb_hbm, idx_hbm, out_hbm, idx_v, rows_v, sem):
        wid = lax.axis_index("s") * 2 + lax.axis_index("c")
        base = wid * _BPW
        pltpu.sync_copy(idx_hbm.at[pl.ds(base, _BPW)], idx_v)
        pltpu.async_copy(cb_hbm.at[idx_v], rows_v, sem).wait()
        pltpu.sync_copy(rows_v, out_hbm.at[pl.ds(base, _BPW)])

    return _gather_rows


def kernel(z_e, codebook):
    cb2, cnorm = _prep_call(codebook)
    idx, loss_sum = _argmin_call(z_e, cb2, cnorm)
    z_q = _make_gather_rows()(codebook, idx)
    vq_loss = (1.0 + _BETA) * loss_sum[0] / (_B * _D)
    return (z_q, idx, vq_loss)


# speedup vs baseline: 3.5325x; 1.0815x over previous
import functools

import jax
import jax.numpy as jnp
from jax import lax
from jax.experimental import pallas as pl
from jax.experimental.pallas import tpu as pltpu
from jax.experimental.pallas import tpu_sc as plsc

_K = 8192
_D = 32
_B = 8192
_BETA = 0.25
_BB = 1024
_W = 2048
_NW = 32
_BPW = _B // _NW


def _rownorm(x):
    sq = x * x
    p = ((sq[:, 8:16] + sq[:, 0:8]) + sq[:, 16:24]) + sq[:, 24:32]
    q = p[:, 4:8] + p[:, 0:4]
    r = q[:, 2:4] + q[:, 0:2]
    return r[:, 1:2] + r[:, 0:1]


def _colnorm(xt):
    sq = xt * xt
    p = ((sq[8:16, :] + sq[0:8, :]) + sq[16:24, :]) + sq[24:32, :]
    q = p[4:8, :] + p[0:4, :]
    r = q[2:4, :] + q[0:2, :]
    return r[1:2, :] + r[0:1, :]


def _prep_body(cb_ref, cb2_ref, cnorm_ref):
    cb = cb_ref[...]
    cb2_ref[...] = cb + cb
    cnorm_ref[...] = _colnorm(cb.T).T


def _prep_call(codebook):
    return pl.pallas_call(
        _prep_body,
        out_shape=[
            jax.ShapeDtypeStruct((_K, _D), jnp.float32),
            jax.ShapeDtypeStruct((_K, 1), jnp.float32),
        ],
    )(codebook)


def _argmin_body(z_ref, cb_ref, idx_ref, loss_ref, cb2_ref, cn_ref):
    @pl.when(pl.program_id(0) == 0)
    def _():
        cb = cb_ref[...]
        cb2_ref[...] = cb + cb
        cn_ref[...] = _colnorm(cb.T).T
        loss_ref[0] = 0.0

    zt = z_ref[...].T
    znorm = _colnorm(zt)
    best_v = best_i = loss_m = None
    for c in range(_K // _W):
        cw2 = cb2_ref[pl.ds(c * _W, _W), :]
        cnorm = cn_ref[pl.ds(c * _W, _W), :]
        zc2 = lax.dot_general(cw2, zt, (((1,), (0,)), ((), ())),
                              preferred_element_type=jnp.float32)
        d2 = (znorm - zc2) + cnorm
        m = jnp.min(d2, axis=0)
        iota = lax.broadcasted_iota(jnp.int32, (_W, _BB), 0)
        il = jnp.min(jnp.where(d2 == m[None, :], iota, _W), axis=0)
        i = il + (c * _W)
        mb = m.astype(jnp.bfloat16).astype(jnp.float32)
        if best_v is None:
            best_v, best_i, loss_m = mb, i, m
        else:
            take = (m < best_v) | ((m == best_v) & (i < best_i))
            best_v = jnp.where(take, mb, best_v)
            best_i = jnp.where(take, i, best_i)
            loss_m = jnp.minimum(loss_m, m)
    idx_ref[...] = best_i
    loss_ref[0] += jnp.sum(loss_m)


def _argmin_call(z_e, codebook):
    return pl.pallas_call(
        _argmin_body,
        grid=(_B // _BB,),
        in_specs=[
            pl.BlockSpec((_BB, _D), lambda i: (i, 0)),
            pl.BlockSpec((_K, _D), lambda i: (0, 0)),
        ],
        scratch_shapes=[
            pltpu.VMEM((_K, _D), jnp.float32),
            pltpu.VMEM((_K, 1), jnp.float32),
        ],
        out_specs=[
            pl.BlockSpec((_BB,), lambda i: (i,)),
            pl.BlockSpec(memory_space=pltpu.SMEM),
        ],
        out_shape=[
            jax.ShapeDtypeStruct((_B,), jnp.int32),
            jax.ShapeDtypeStruct((1,), jnp.float32),
        ],
        compiler_params=pltpu.CompilerParams(
            dimension_semantics=("arbitrary",)),
    )(z_e, codebook)


@functools.cache
def _make_gather_rows():
    @functools.partial(
        pl.kernel,
        mesh=plsc.VectorSubcoreMesh(core_axis_name="c", subcore_axis_name="s"),
        out_type=jax.ShapeDtypeStruct((_B, _D), jnp.float32),
        scratch_types=[
            pltpu.VMEM((_BPW,), jnp.int32),
            pltpu.VMEM((_BPW, _D), jnp.float32),
            pltpu.SemaphoreType.DMA,
        ],
        compiler_params=pltpu.CompilerParams(use_tc_tiling_on_sc=False),
    )
    def _gather_rows(cb_hbm, idx_hbm, out_hbm, idx_v, rows_v, sem):
        wid = lax.axis_index("s") * 2 + lax.axis_index("c")
        base = wid * _BPW
        pltpu.sync_copy(idx_hbm.at[pl.ds(base, _BPW)], idx_v)
        pltpu.async_copy(cb_hbm.at[idx_v], rows_v, sem).wait()
        pltpu.sync_copy(rows_v, out_hbm.at[pl.ds(base, _BPW)])

    return _gather_rows


def kernel(z_e, codebook):
    idx, loss_sum = _argmin_call(z_e, codebook)
    z_q = _make_gather_rows()(codebook, idx)
    vq_loss = (1.0 + _BETA) * loss_sum[0] / (_B * _D)
    return (z_q, idx, vq_loss)


# speedup vs baseline: 3.5390x; 1.0019x over previous
import functools

import jax
import jax.numpy as jnp
from jax import lax
from jax.experimental import pallas as pl
from jax.experimental.pallas import tpu as pltpu
from jax.experimental.pallas import tpu_sc as plsc

_K = 8192
_D = 32
_B = 8192
_BETA = 0.25
_BB = 2048
_W = 2048
_NW = 32
_BPW = _B // _NW


def _rownorm(x):
    sq = x * x
    p = ((sq[:, 8:16] + sq[:, 0:8]) + sq[:, 16:24]) + sq[:, 24:32]
    q = p[:, 4:8] + p[:, 0:4]
    r = q[:, 2:4] + q[:, 0:2]
    return r[:, 1:2] + r[:, 0:1]


def _colnorm(xt):
    sq = xt * xt
    p = ((sq[8:16, :] + sq[0:8, :]) + sq[16:24, :]) + sq[24:32, :]
    q = p[4:8, :] + p[0:4, :]
    r = q[2:4, :] + q[0:2, :]
    return r[1:2, :] + r[0:1, :]


def _prep_body(cb_ref, cb2_ref, cnorm_ref):
    cb = cb_ref[...]
    cb2_ref[...] = cb + cb
    cnorm_ref[...] = _colnorm(cb.T).T


def _prep_call(codebook):
    return pl.pallas_call(
        _prep_body,
        out_shape=[
            jax.ShapeDtypeStruct((_K, _D), jnp.float32),
            jax.ShapeDtypeStruct((_K, 1), jnp.float32),
        ],
    )(codebook)


def _argmin_body(z_ref, cb_ref, idx_ref, loss_ref, cb2_ref, cn_ref):
    @pl.when(pl.program_id(0) == 0)
    def _():
        cb = cb_ref[...]
        cb2_ref[...] = cb + cb
        cn_ref[...] = _colnorm(cb.T).T
        loss_ref[0] = 0.0

    zt = z_ref[...].T
    znorm = _colnorm(zt)
    best_v = best_i = loss_m = None
    for c in range(_K // _W):
        cw2 = cb2_ref[pl.ds(c * _W, _W), :]
        cnorm = cn_ref[pl.ds(c * _W, _W), :]
        zc2 = lax.dot_general(cw2, zt, (((1,), (0,)), ((), ())),
                              preferred_element_type=jnp.float32)
        d2 = (znorm - zc2) + cnorm
        m = jnp.min(d2, axis=0)
        iota = lax.broadcasted_iota(jnp.int32, (_W, _BB), 0)
        il = jnp.min(jnp.where(d2 == m[None, :], iota, _W), axis=0)
        i = il + (c * _W)
        mb = m.astype(jnp.bfloat16).astype(jnp.float32)
        if best_v is None:
            best_v, best_i, loss_m = mb, i, m
        else:
            take = (m < best_v) | ((m == best_v) & (i < best_i))
            best_v = jnp.where(take, mb, best_v)
            best_i = jnp.where(take, i, best_i)
            loss_m = jnp.minimum(loss_m, m)
    idx_ref[...] = best_i
    loss_ref[0] += jnp.sum(loss_m)


def _argmin_call(z_e, codebook):
    return pl.pallas_call(
        _argmin_body,
        grid=(_B // _BB,),
        in_specs=[
            pl.BlockSpec((_BB, _D), lambda i: (i, 0)),
            pl.BlockSpec((_K, _D), lambda i: (0, 0)),
        ],
        scratch_shapes=[
            pltpu.VMEM((_K, _D), jnp.float32),
            pltpu.VMEM((_K, 1), jnp.float32),
        ],
        out_specs=[
            pl.BlockSpec((_BB,), lambda i: (i,)),
            pl.BlockSpec(memory_space=pltpu.SMEM),
        ],
        out_shape=[
            jax.ShapeDtypeStruct((_B,), jnp.int32),
            jax.ShapeDtypeStruct((1,), jnp.float32),
        ],
        compiler_params=pltpu.CompilerParams(
            dimension_semantics=("arbitrary",)),
    )(z_e, codebook)


@functools.cache
def _make_gather_rows():
    @functools.partial(
        pl.kernel,
        mesh=plsc.VectorSubcoreMesh(core_axis_name="c", subcore_axis_name="s"),
        out_type=jax.ShapeDtypeStruct((_B, _D), jnp.float32),
        scratch_types=[
            pltpu.VMEM((_BPW,), jnp.int32),
            pltpu.VMEM((_BPW, _D), jnp.float32),
            pltpu.SemaphoreType.DMA,
        ],
        compiler_params=pltpu.CompilerParams(use_tc_tiling_on_sc=False),
    )
    def _gather_rows(cb_hbm, idx_hbm, out_hbm, idx_v, rows_v, sem):
        wid = lax.axis_index("s") * 2 + lax.axis_index("c")
        base = wid * _BPW
        pltpu.sync_copy(idx_hbm.at[pl.ds(base, _BPW)], idx_v)
        pltpu.async_copy(cb_hbm.at[idx_v], rows_v, sem).wait()
        pltpu.sync_copy(rows_v, out_hbm.at[pl.ds(base, _BPW)])

    return _gather_rows


def kernel(z_e, codebook):
    idx, loss_sum = _argmin_call(z_e, codebook)
    z_q = _make_gather_rows()(codebook, idx)
    vq_loss = (1.0 + _BETA) * loss_sum[0] / (_B * _D)
    return (z_q, idx, vq_loss)
